# Initial kernel scaffold; baseline (speedup 1.0000x reference)
import jax
import jax.numpy as jnp
from jax.experimental import pallas as pl


def kernel(x, Wg, bg, W1, b1, W2, b2):
    raise NotImplementedError("write your pallas kernel here")


# speedup vs baseline: 1.7436x; 1.7436x over previous
import functools

import jax
import jax.numpy as jnp
from jax.experimental import pallas as pl
from jax.experimental.pallas import tpu as pltpu

TEMP = 2.718281828459045
NEG_INF = -1e30


def _moe_body(x_ref, Wg_ref, bg_ref, b1_ref, b2_ref, W1_ref, W2_ref,
              o_ref, w_ref, *, na, bt):
    e = pl.program_id(1)
    E = Wg_ref.shape[0]

    @pl.when(e == 0)
    def _gate():
        x = x_ref[...]
        logits = jax.lax.dot_general(
            x, Wg_ref[...], (((1,), (1,)), ((), ())),
            preferred_element_type=jnp.float32)
        logits = logits + bg_ref[...]
        scaled = logits / TEMP
        m = jnp.max(scaled, axis=-1, keepdims=True)
        ex = jnp.exp(scaled - m)
        p = ex / jnp.sum(ex, axis=-1, keepdims=True)
        iota = jax.lax.broadcasted_iota(jnp.int32, (bt, E), 1)
        work = p
        mask = jnp.zeros((bt, E), dtype=jnp.float32)
        for _ in range(na):
            mx = jnp.max(work, axis=-1, keepdims=True)
            cand = jnp.where(work == mx, iota, E)
            sel = jnp.min(cand, axis=-1, keepdims=True)
            onehot = (iota == sel).astype(jnp.float32)
            mask = mask + onehot
            work = jnp.where(onehot > 0, NEG_INF, work)
        w = p * mask
        w = w / (jnp.sum(w, axis=-1, keepdims=True) + 1e-8)
        w_ref[...] = w
        o_ref[...] = jax.lax.dot_general(
            w, b2_ref[...], (((1,), (0,)), ((), ())),
            preferred_element_type=jnp.float32)

    x = x_ref[...]
    h = jax.lax.dot_general(
        x, W1_ref[0], (((1,), (1,)), ((), ())),
        preferred_element_type=jnp.float32)
    h = jnp.maximum(h + b1_ref[0], 0.0)
    o = jax.lax.dot_general(
        h, W2_ref[0], (((1,), (1,)), ((), ())),
        preferred_element_type=jnp.float32)
    lane = jax.lax.broadcasted_iota(jnp.int32, (bt, E), 1)
    w_col = jnp.sum(
        jnp.where(lane == e, w_ref[...], 0.0), axis=-1, keepdims=True)
    o_ref[...] += w_col * o


def kernel(x, Wg, bg, W1, b1, W2, b2):
    T, D = x.shape
    E, H, _ = W1.shape
    O = W2.shape[1]
    na = max(1, int(E * 0.7))
    bt = min(1024, T)
    grid = (T // bt, E)

    body = functools.partial(_moe_body, na=na, bt=bt)
    out = pl.pallas_call(
        body,
        grid=grid,
        in_specs=[
            pl.BlockSpec((bt, D), lambda t, e: (t, 0)),
            pl.BlockSpec((E, D), lambda t, e: (0, 0)),
            pl.BlockSpec((1, E), lambda t, e: (0, 0)),
            pl.BlockSpec((1, 1, H), lambda t, e: (e, 0, 0)),
            pl.BlockSpec((E, O), lambda t, e: (0, 0)),
            pl.BlockSpec((1, H, D), lambda t, e: (e, 0, 0)),
            pl.BlockSpec((1, O, H), lambda t, e: (e, 0, 0)),
        ],
        out_specs=pl.BlockSpec((bt, O), lambda t, e: (t, 0)),
        out_shape=jax.ShapeDtypeStruct((T, O), jnp.float32),
        scratch_shapes=[pltpu.VMEM((bt, E), jnp.float32)],
        compiler_params=pltpu.CompilerParams(
            dimension_semantics=("parallel", "arbitrary")),
    )(x, Wg, bg.reshape(1, E), b1.reshape(E, 1, H), b2, W1, W2)
    return out


# speedup vs baseline: 1.8161x; 1.0416x over previous
import functools

import jax
import jax.numpy as jnp
from jax.experimental import pallas as pl
from jax.experimental.pallas import tpu as pltpu

TEMP = 2.718281828459045
NEG_INF = -1e30


def _moe_body(x_ref, Wg_ref, bg_ref, b1_ref, b2_ref, W1_ref, W2_ref,
              o_ref, w_ref, *, na, bt):
    e = pl.program_id(1)
    E = Wg_ref.shape[0]

    @pl.when(e == 0)
    def _gate():
        x = x_ref[...]
        logits = jax.lax.dot_general(
            x, Wg_ref[...], (((1,), (1,)), ((), ())),
            preferred_element_type=jnp.float32)
        logits_t = jnp.transpose(logits) + bg_ref[...]
        scaled = logits_t / TEMP
        m = jnp.max(scaled, axis=0, keepdims=True)
        ex = jnp.exp(scaled - m)
        p = ex / jnp.sum(ex, axis=0, keepdims=True)
        iota = jax.lax.broadcasted_iota(jnp.int32, (E, bt), 0)
        work = p
        mask = jnp.zeros((E, bt), dtype=jnp.float32)
        for _ in range(na):
            mx = jnp.max(work, axis=0, keepdims=True)
            cand = jnp.where(work == mx, iota, E)
            sel = jnp.min(cand, axis=0, keepdims=True)
            onehot = (iota == sel).astype(jnp.float32)
            mask = mask + onehot
            work = jnp.where(onehot > 0, NEG_INF, work)
        w_t = p * mask
        w_t = w_t / (jnp.sum(w_t, axis=0, keepdims=True) + 1e-8)
        w = jnp.transpose(w_t)
        w_ref[...] = w
        o_ref[...] = jax.lax.dot_general(
            w, b2_ref[...], (((1,), (0,)), ((), ())),
            preferred_element_type=jnp.float32)

    x = x_ref[...]
    h = jax.lax.dot_general(
        x, W1_ref[0], (((1,), (1,)), ((), ())),
        preferred_element_type=jnp.float32)
    h = jnp.maximum(h + b1_ref[0], 0.0)
    o = jax.lax.dot_general(
        h, W2_ref[0], (((1,), (1,)), ((), ())),
        preferred_element_type=jnp.float32)
    lane = jax.lax.broadcasted_iota(jnp.int32, (bt, E), 1)
    w_col = jnp.sum(
        jnp.where(lane == e, w_ref[...], 0.0), axis=-1, keepdims=True)
    o_ref[...] += w_col * o


def kernel(x, Wg, bg, W1, b1, W2, b2):
    T, D = x.shape
    E, H, _ = W1.shape
    O = W2.shape[1]
    na = max(1, int(E * 0.7))
    bt = min(1024, T)
    grid = (T // bt, E)

    body = functools.partial(_moe_body, na=na, bt=bt)
    out = pl.pallas_call(
        body,
        grid=grid,
        in_specs=[
            pl.BlockSpec((bt, D), lambda t, e: (t, 0)),
            pl.BlockSpec((E, D), lambda t, e: (0, 0)),
            pl.BlockSpec((E, 1), lambda t, e: (0, 0)),
            pl.BlockSpec((1, 1, H), lambda t, e: (e, 0, 0)),
            pl.BlockSpec((E, O), lambda t, e: (0, 0)),
            pl.BlockSpec((1, H, D), lambda t, e: (e, 0, 0)),
            pl.BlockSpec((1, O, H), lambda t, e: (e, 0, 0)),
        ],
        out_specs=pl.BlockSpec((bt, O), lambda t, e: (t, 0)),
        out_shape=jax.ShapeDtypeStruct((T, O), jnp.float32),
        scratch_shapes=[pltpu.VMEM((bt, E), jnp.float32)],
        compiler_params=pltpu.CompilerParams(
            dimension_semantics=("parallel", "arbitrary")),
    )(x, Wg, bg.reshape(E, 1), b1.reshape(E, 1, H), b2, W1, W2)
    return out


# speedup vs baseline: 1.8307x; 1.0080x over previous
import functools

import jax
import jax.numpy as jnp
from jax.experimental import pallas as pl
from jax.experimental.pallas import tpu as pltpu

TEMP = 2.718281828459045
NEG_INF = -1e30


def _moe_body(x_ref, Wg_ref, bg_ref, b1_ref, b2_ref, W1_ref, W2_ref,
              o_ref, w_ref, *, na, bt):
    e = pl.program_id(1)
    E = Wg_ref.shape[0]

    @pl.when(e == 0)
    def _gate():
        x = x_ref[...]
        logits = jax.lax.dot_general(
            x, Wg_ref[...], (((1,), (1,)), ((), ())),
            preferred_element_type=jnp.float32)
        logits_t = jnp.transpose(logits) + bg_ref[...]
        scaled = logits_t / TEMP
        m = jnp.max(scaled, axis=0, keepdims=True)
        ex = jnp.exp(scaled - m)
        p = ex / jnp.sum(ex, axis=0, keepdims=True)
        iota = jax.lax.broadcasted_iota(jnp.int32, (E, bt), 0)
        work = p
        mask = jnp.zeros((E, bt), dtype=jnp.float32)
        for _ in range(na):
            mx = jnp.max(work, axis=0, keepdims=True)
            cand = jnp.where(work == mx, iota, E)
            sel = jnp.min(cand, axis=0, keepdims=True)
            onehot = (iota == sel).astype(jnp.float32)
            mask = mask + onehot
            work = jnp.where(onehot > 0, NEG_INF, work)
        w_t = p * mask
        w_t = w_t / (jnp.sum(w_t, axis=0, keepdims=True) + 1e-8)
        w = jnp.transpose(w_t)
        w_ref[...] = w
        o_ref[...] = jax.lax.dot_general(
            w, b2_ref[...], (((1,), (0,)), ((), ())),
            preferred_element_type=jnp.float32)

    x = x_ref[...].astype(jnp.bfloat16)
    h = jax.lax.dot_general(
        x, W1_ref[0].astype(jnp.bfloat16), (((1,), (1,)), ((), ())),
        preferred_element_type=jnp.float32)
    h = jnp.maximum(h + b1_ref[0], 0.0).astype(jnp.bfloat16)
    o = jax.lax.dot_general(
        h, W2_ref[0].astype(jnp.bfloat16), (((1,), (1,)), ((), ())),
        preferred_element_type=jnp.float32)
    lane = jax.lax.broadcasted_iota(jnp.int32, (bt, E), 1)
    w_col = jnp.sum(
        jnp.where(lane == e, w_ref[...], 0.0), axis=-1, keepdims=True)
    o_ref[...] += w_col * o


def kernel(x, Wg, bg, W1, b1, W2, b2):
    T, D = x.shape
    E, H, _ = W1.shape
    O = W2.shape[1]
    na = max(1, int(E * 0.7))
    bt = min(1024, T)
    grid = (T // bt, E)

    body = functools.partial(_moe_body, na=na, bt=bt)
    out = pl.pallas_call(
        body,
        grid=grid,
        in_specs=[
            pl.BlockSpec((bt, D), lambda t, e: (t, 0)),
            pl.BlockSpec((E, D), lambda t, e: (0, 0)),
            pl.BlockSpec((E, 1), lambda t, e: (0, 0)),
            pl.BlockSpec((1, 1, H), lambda t, e: (e, 0, 0)),
            pl.BlockSpec((E, O), lambda t, e: (0, 0)),
            pl.BlockSpec((1, H, D), lambda t, e: (e, 0, 0)),
            pl.BlockSpec((1, O, H), lambda t, e: (e, 0, 0)),
        ],
        out_specs=pl.BlockSpec((bt, O), lambda t, e: (t, 0)),
        out_shape=jax.ShapeDtypeStruct((T, O), jnp.float32),
        scratch_shapes=[pltpu.VMEM((bt, E), jnp.float32)],
        compiler_params=pltpu.CompilerParams(
            dimension_semantics=("parallel", "arbitrary")),
    )(x, Wg, bg.reshape(E, 1), b1.reshape(E, 1, H), b2, W1, W2)
    return out
